# baseline (speedup 1.0000x reference)
import functools
import math

import jax
import jax.numpy as jnp
import numpy as np
from jax import lax
from jax.experimental import pallas as pl
from jax.experimental.pallas import tpu as pltpu
from jax.experimental.pallas import tpu_sc as plsc

N = 10000
E = 320000
H = 8
D = 16
HC = 128
NRBF = 32
CUTOFF_UPPER = 5.0

BN = 200
BE = 1280

_EPS = 1e-6


def _head_mask():
    r = jax.lax.broadcasted_iota(jnp.int32, (HC, H), 0) // D
    c = jax.lax.broadcasted_iota(jnp.int32, (HC, H), 1)
    return (r == c).astype(jnp.float32)


def _head_mask64():
    r = jax.lax.broadcasted_iota(jnp.int32, (64, H), 0) // (D // 2)
    c = jax.lax.broadcasted_iota(jnp.int32, (64, H), 1)
    return (r == c).astype(jnp.float32)


def _silu(x):
    return x * jax.nn.sigmoid(x)


def _node_body(x_ref, vec_ref, Wq_ref, bq_ref, Wk_ref, bk_ref, Wv_ref, bv_ref,
               Wvec_ref, lns_ref, lnb_ref, lnqs_ref, lnqb_ref, lnks_ref,
               lnkb_ref, q_ref, k_ref, v_ref, vd_ref, v3_ref):
    xb = x_ref[...]
    mean = jnp.mean(xb, axis=1, keepdims=True)
    xc = xb - mean
    var = jnp.mean(xc * xc, axis=1, keepdims=True)
    xn = xc * jax.lax.rsqrt(var + _EPS) * lns_ref[...] + lnb_ref[...]

    MH = _head_mask()

    def headln(z, s, b):
        m = (z @ MH) * (1.0 / D)
        mb = m @ MH.T
        zc = z - mb
        v2 = ((zc * zc) @ MH) * (1.0 / D)
        vb = v2 @ MH.T
        return zc * jax.lax.rsqrt(vb + _EPS) * s + b

    q_ref[...] = headln(xn @ Wq_ref[...] + bq_ref[...], lnqs_ref[...],
                        lnqb_ref[...])
    k_ref[...] = headln(xn @ Wk_ref[...] + bk_ref[...], lnks_ref[...],
                        lnkb_ref[...])
    v_ref[...] = (xn @ Wv_ref[...] + bv_ref[...]).astype(jnp.bfloat16)

    vecb = vec_ref[...]
    Wvec = Wvec_ref[...]
    acc = jnp.zeros((vecb.shape[0], HC), jnp.float32)
    for c in range(3):
        p = vecb[:, c * HC:(c + 1) * HC] @ Wvec
        acc = acc + p[:, :HC] * p[:, HC:2 * HC]
        v3_ref[:, c * HC:(c + 1) * HC] = p[:, 2 * HC:]
    vd_ref[...] = acc


def _node_stage(x, vecf, Wq, bq, Wk, bk, Wv_p, bv_p, Wvec_d, ln_s, ln_b,
                lnq_s, lnq_b, lnk_s, lnk_b):
    nb = N // BN
    row = lambda i: (i, 0)
    rep = lambda i: (0, 0)
    out_shapes = (
        jax.ShapeDtypeStruct((N, HC), jnp.float32),
        jax.ShapeDtypeStruct((N, HC), jnp.float32),
        jax.ShapeDtypeStruct((N, 3 * HC), jnp.bfloat16),
        jax.ShapeDtypeStruct((N, HC), jnp.float32),
        jax.ShapeDtypeStruct((N, 3 * HC), jnp.float32),
    )
    in_specs = [
        pl.BlockSpec((BN, HC), row),
        pl.BlockSpec((BN, 3 * HC), row),
        pl.BlockSpec((HC, HC), rep),
        pl.BlockSpec((1, HC), rep),
        pl.BlockSpec((HC, HC), rep),
        pl.BlockSpec((1, HC), rep),
        pl.BlockSpec((HC, 3 * HC), rep),
        pl.BlockSpec((1, 3 * HC), rep),
        pl.BlockSpec((HC, 3 * HC), rep),
        pl.BlockSpec((1, HC), rep),
        pl.BlockSpec((1, HC), rep),
        pl.BlockSpec((1, HC), rep),
        pl.BlockSpec((1, HC), rep),
        pl.BlockSpec((1, HC), rep),
        pl.BlockSpec((1, HC), rep),
    ]
    out_specs = (
        pl.BlockSpec((BN, HC), row),
        pl.BlockSpec((BN, HC), row),
        pl.BlockSpec((BN, 3 * HC), row),
        pl.BlockSpec((BN, HC), row),
        pl.BlockSpec((BN, 3 * HC), row),
    )
    return pl.pallas_call(
        _node_body, grid=(nb,), in_specs=in_specs, out_specs=out_specs,
        out_shape=out_shapes,
    )(x, vecf, Wq, bq, Wk, bk, Wv_p, bv_p, Wvec_d, ln_s, ln_b, lnq_s, lnq_b,
      lnk_s, lnk_b)


NC = 2
NS = 16
NW = NC * NS
GCH = 80
KVV = 128 + 192 + 192


def _sc_gather(q32, kvv32, senders, receivers, e0, es):
    epw = es // NW
    nch = epw // GCH
    mesh = plsc.VectorSubcoreMesh(core_axis_name="c", subcore_axis_name="s")

    @functools.partial(
        pl.kernel, mesh=mesh,
        out_type=(
            jax.ShapeDtypeStruct((es, HC), jnp.int32),
            jax.ShapeDtypeStruct((es, KVV), jnp.int32),
        ),
        scratch_types=[
            pltpu.VMEM((2, GCH), jnp.int32),
            pltpu.VMEM((2, GCH), jnp.int32),
            pltpu.VMEM((GCH, HC), jnp.int32),
            pltpu.VMEM((GCH, HC), jnp.int32),
            pltpu.VMEM((GCH, KVV), jnp.int32),
            pltpu.VMEM((GCH, KVV), jnp.int32),
            pltpu.SemaphoreType.DMA,
            pltpu.SemaphoreType.DMA,
            pltpu.SemaphoreType.DMA,
            pltpu.SemaphoreType.DMA,
            pltpu.SemaphoreType.DMA,
            pltpu.SemaphoreType.DMA,
        ],
    )
    def gk(q_hbm, kvv_hbm, send_hbm, recv_hbm, qi_hbm, kvvj_hbm,
           idx_r, idx_s, ra0, ra1, rb0, rb1, semi0, semi1, semg0, semg1,
           semw0, semw1):
        wid = lax.axis_index("s") * NC + lax.axis_index("c")
        base0 = wid * epw
        ra = (ra0, ra1)
        rb = (rb0, rb1)
        semi = (semi0, semi1)
        semg = (semg0, semg1)
        semw = (semw0, semw1)

        def body(g, _):
            i0 = 2 * g
            bs = []
            for p in range(2):
                b = pl.multiple_of(base0 + (i0 + p) * GCH, 8)
                src = pl.multiple_of(e0 + b, 8)
                bs.append(b)
                pltpu.async_copy(recv_hbm.at[pl.ds(src, GCH)],
                                 idx_r.at[p], semi[p])
                pltpu.async_copy(send_hbm.at[pl.ds(src, GCH)],
                                 idx_s.at[p], semi[p])
            for p in range(2):
                pltpu.make_async_copy(
                    recv_hbm.at[pl.ds(0, GCH)], idx_r.at[p], semi[p]).wait()
                pltpu.make_async_copy(
                    send_hbm.at[pl.ds(0, GCH)], idx_s.at[p], semi[p]).wait()
                pltpu.async_copy(q_hbm.at[idx_r.at[p]], ra[p], semg[p])
                pltpu.async_copy(kvv_hbm.at[idx_s.at[p]], rb[p], semg[p])
            for p in range(2):
                pltpu.make_async_copy(
                    q_hbm.at[idx_r.at[p]], ra[p], semg[p]).wait()
                pltpu.make_async_copy(
                    kvv_hbm.at[idx_s.at[p]], rb[p], semg[p]).wait()
                pltpu.async_copy(ra[p], qi_hbm.at[pl.ds(bs[p], GCH)],
                                 semw[p])
                pltpu.async_copy(rb[p], kvvj_hbm.at[pl.ds(bs[p], GCH)],
                                 semw[p])
            for p in range(2):
                pltpu.make_async_copy(
                    ra[p], qi_hbm.at[pl.ds(bs[p], GCH)], semw[p]).wait()
                pltpu.make_async_copy(
                    rb[p], kvvj_hbm.at[pl.ds(bs[p], GCH)], semw[p]).wait()
            return ()

        lax.fori_loop(0, nch // 2, body, (), unroll=False)

        if nch % 2:
            b = pl.multiple_of(base0 + (nch - 1) * GCH, 8)
            src = pl.multiple_of(e0 + b, 8)
            pltpu.sync_copy(recv_hbm.at[pl.ds(src, GCH)], idx_r.at[0])
            pltpu.sync_copy(send_hbm.at[pl.ds(src, GCH)], idx_s.at[0])
            pltpu.async_copy(q_hbm.at[idx_r.at[0]], ra0, semg0).wait()
            pltpu.async_copy(kvv_hbm.at[idx_s.at[0]], rb0, semg0).wait()
            pltpu.sync_copy(ra0, qi_hbm.at[pl.ds(b, GCH)])
            pltpu.sync_copy(rb0, kvvj_hbm.at[pl.ds(b, GCH)])

    return gk(q32, kvv32, senders, receivers)


def _unpk(x):
    e = jax.lax.bitcast_convert_type(x << 16, jnp.float32)
    o = jax.lax.bitcast_convert_type(x & jnp.int32(-65536), jnp.float32)
    return e, o


def _msg_body(qi_ref, kvv_ref, ea_ref, evw_ref, Wdk_ref, bdk_ref, Wdv_ref,
              bdv_ref, m_ref):
    ea = ea_ref[...]
    dk = _silu(ea @ Wdk_ref[...] + bdk_ref[...])
    dv = _silu(ea @ Wdv_ref[...] + bdv_ref[...])

    evw = evw_ref[...]
    w = evw[:, 3:4]
    cut = 0.5 * (jnp.cos(w * (math.pi / CUTOFF_UPPER)) + 1.0)
    cut = cut * (w < CUTOFF_UPPER).astype(jnp.float32)

    qi = jax.lax.bitcast_convert_type(qi_ref[...], jnp.float32)
    kvv = kvv_ref[...]
    kj = jax.lax.bitcast_convert_type(kvv[:, :HC], jnp.float32)
    v0, v1 = _unpk(kvv[:, HC:2 * HC])
    v2, c0 = _unpk(kvv[:, 2 * HC:3 * HC])
    c1, c2 = _unpk(kvv[:, 3 * HC:])

    MH = _head_mask()
    t = qi * kj * dk
    attn = _silu(t @ MH) * cut
    ae = attn @ MH.T

    m_ref[0] = v0 * dv[:, :HC] * ae
    A = v1 * dv[:, HC:2 * HC]
    B = v2 * dv[:, 2 * HC:]
    for c, cc in enumerate((c0, c1, c2)):
        m_ref[c + 1] = cc * A + B * evw[:, c:c + 1]


def _msg_stage(qi32, kvvj, edge_attr, evw, Wdk_d, bdk_d, Wdv_d, bdv_d,
               e0, es):
    nb = es // BE
    off = e0 // BE
    row = lambda i: (i, 0)
    erow = lambda i: (i + off, 0)
    rep = lambda i: (0, 0)
    in_specs = [
        pl.BlockSpec((BE, HC), row),
        pl.BlockSpec((BE, KVV), row),
        pl.BlockSpec((BE, NRBF), erow),
        pl.BlockSpec((BE, 4), erow),
        pl.BlockSpec((NRBF, HC), rep),
        pl.BlockSpec((1, HC), rep),
        pl.BlockSpec((NRBF, 3 * HC), rep),
        pl.BlockSpec((1, 3 * HC), rep),
    ]
    out_spec = pl.BlockSpec((4, BE, HC), lambda i: (0, i, 0))
    return pl.pallas_call(
        _msg_body, grid=(nb,), in_specs=in_specs, out_specs=out_spec,
        out_shape=jax.ShapeDtypeStruct((4, es, HC), jnp.float32),
    )(qi32, kvvj, edge_attr, evw, Wdk_d, bdk_d, Wdv_d, bdv_d)


SCH = 80
NPAD = 10240
NPT = NPAD // NS


def _sc_scatter(m_flat, receivers, zeros, e0, es):
    ept = es // NS
    nch = ept // SCH
    mesh = plsc.VectorSubcoreMesh(core_axis_name="c", subcore_axis_name="s")

    assert nch % 2 == 0

    @functools.partial(
        pl.kernel, mesh=mesh,
        out_type=jax.ShapeDtypeStruct((4 * NPAD, HC), jnp.float32),
        scratch_types=[
            pltpu.VMEM_SHARED((NPAD, HC), jnp.float32),
            pltpu.VMEM((SCH,), jnp.int32),
            pltpu.VMEM((SCH,), jnp.int32),
            pltpu.VMEM((SCH, HC), jnp.float32),
            pltpu.VMEM((SCH, HC), jnp.float32),
            pltpu.SemaphoreType.DMA,
            pltpu.SemaphoreType.DMA,
            pltpu.SemaphoreType.DMA,
            pltpu.SemaphoreType.DMA,
        ],
    )
    def sk(m_hbm, recv_hbm, z_hbm, s_hbm, shared, idx0, idx1, rows0, rows1,
           semi0, semi1, semr0, semr1):
        cid = lax.axis_index("c")
        sid = lax.axis_index("s")
        r0 = sid * NPT
        eb0 = sid * ept
        idx = (idx0, idx1)
        rows = (rows0, rows1)
        semi = (semi0, semi1)
        semr = (semr0, semr1)
        for j in range(2):
            ch = cid * 2 + j
            pltpu.sync_copy(z_hbm, shared.at[pl.ds(r0, NPT)])
            plsc.subcore_barrier()

            def start(p, i):
                b = pl.multiple_of(eb0 + i * SCH, 8)
                pltpu.async_copy(
                    recv_hbm.at[pl.ds(pl.multiple_of(e0 + b, 8), SCH)],
                    idx[p], semi[p])
                pltpu.async_copy(
                    m_hbm.at[pl.ds(pl.multiple_of(ch * es + b, 8), SCH)],
                    rows[p], semr[p])

            def fin(p):
                pltpu.make_async_copy(
                    recv_hbm.at[pl.ds(0, SCH)], idx[p], semi[p]).wait()
                pltpu.make_async_copy(
                    m_hbm.at[pl.ds(0, SCH)], rows[p], semr[p]).wait()
                pltpu.sync_copy(rows[p], shared.at[idx[p]], add=True)

            start(0, 0)

            def body(g, _):
                start(1, 2 * g + 1)
                fin(0)
                start(0, 2 * g + 2)
                fin(1)
                return ()

            lax.fori_loop(0, nch // 2 - 1, body, (), unroll=False)
            start(1, nch - 1)
            fin(0)
            fin(1)
            plsc.subcore_barrier()
            pltpu.sync_copy(shared.at[pl.ds(r0, NPT)],
                            s_hbm.at[pl.ds(ch * NPAD + r0, NPT)])
            plsc.subcore_barrier()

    return sk(m_flat, receivers, zeros)


def _out_body(sa_ref, sb_ref, sc_ref, vd_ref, v3_ref, Wo_ref, bo_ref,
              dx_ref, dvec_ref):
    s = sa_ref[...] + sb_ref[...] + sc_ref[...]
    o = s[0] @ Wo_ref[...] + bo_ref[...]
    o1, o2, o3 = o[:, :HC], o[:, HC:2 * HC], o[:, 2 * HC:]
    dx_ref[...] = vd_ref[...] * o2 + o3
    v3 = v3_ref[...]
    for c in range(3):
        dvec_ref[:, c * HC:(c + 1) * HC] = v3[:, c * HC:(c + 1) * HC] * o1 \
            + s[c + 1]


def _out_stage(Sa, Sb, Sc, vec_dot, vec3, Wo_d, bo_d):
    nb = N // BN
    row = lambda i: (i, 0)
    rep = lambda i: (0, 0)
    in_specs = [
        pl.BlockSpec((4, BN, HC), lambda i: (0, i, 0)),
        pl.BlockSpec((4, BN, HC), lambda i: (0, i, 0)),
        pl.BlockSpec((4, BN, HC), lambda i: (0, i, 0)),
        pl.BlockSpec((BN, HC), row),
        pl.BlockSpec((BN, 3 * HC), row),
        pl.BlockSpec((HC, 3 * HC), rep),
        pl.BlockSpec((1, 3 * HC), rep),
    ]
    out_specs = (
        pl.BlockSpec((BN, HC), row),
        pl.BlockSpec((BN, 3 * HC), row),
    )
    return pl.pallas_call(
        _out_body, grid=(nb,), in_specs=in_specs, out_specs=out_specs,
        out_shape=(jax.ShapeDtypeStruct((N, HC), jnp.float32),
                   jax.ShapeDtypeStruct((N, 3 * HC), jnp.float32)),
    )(Sa, Sb, Sc, vec_dot, vec3, Wo_d, bo_d)


def _vperm():
    perm = np.empty((3 * HC,), np.int32)
    for p in range(3):
        for h in range(H):
            for d in range(D):
                perm[p * HC + h * D + d] = h * 3 * D + p * D + d
    return perm


def _pack2(a, b):
    ab = jnp.stack([a, b], axis=-1).reshape(a.shape[0], 2 * a.shape[1])
    return _pack32(ab)


def _pack32(a):
    return jax.lax.bitcast_convert_type(
        a.reshape(a.shape[0], a.shape[1] // 2, 2), jnp.int32)


def kernel(x, vec, edge_weight, edge_attr, edge_vec, ln_s, ln_b, Wq, bq, Wk,
           bk, lnq_s, lnq_b, lnk_s, lnk_b, Wv, bv, Wvec, Wdk, bdk, Wdv, bdv,
           Wo, bo, senders, receivers):
    perm = _vperm()
    Wv_p = Wv[:, perm]
    bv_p = bv[perm].reshape(1, -1)
    Wdk_d = Wdk
    bdk_d = bdk.reshape(1, -1)
    Wdv_d = Wdv[:, perm]
    bdv_d = bdv[perm].reshape(1, -1)

    vecf = vec.reshape(N, 3 * HC)
    evw = jnp.concatenate([edge_vec, edge_weight], axis=1)
    r1 = lambda a: a.reshape(1, -1)
    tile8 = lambda a: jnp.tile(a, (H,)).reshape(1, -1)

    q, k, v, vec_dot, vec3 = _node_stage(
        x, vecf, Wq, r1(bq), Wk, r1(bk), Wv_p, bv_p, Wvec, r1(ln_s),
        r1(ln_b), tile8(lnq_s), tile8(lnq_b), tile8(lnk_s), tile8(lnk_b))

    vb = vecf.astype(jnp.bfloat16)
    q32 = jax.lax.bitcast_convert_type(q, jnp.int32)
    kvv32 = jnp.concatenate(
        [jax.lax.bitcast_convert_type(k, jnp.int32),
         _pack2(v[:, :HC], v[:, HC:2 * HC]),
         _pack2(v[:, 2 * HC:], vb[:, :HC]),
         _pack2(vb[:, HC:2 * HC], vb[:, 2 * HC:])], axis=1)

    zeros = jnp.zeros((NPT, HC), jnp.float32)
    Ss = []
    for e0, es in ((0, 64000), (64000, 128000), (192000, 128000)):
        qi32, kvvj = _sc_gather(q32, kvv32, senders, receivers, e0, es)
        M = _msg_stage(qi32, kvvj, edge_attr, evw, Wdk_d, bdk_d, Wdv_d,
                       bdv_d, e0, es)
        Ss.append(_sc_scatter(M.reshape(4 * es, HC), receivers, zeros,
                              e0, es).reshape(4, NPAD, HC))

    dx, dvecf = _out_stage(Ss[0], Ss[1], Ss[2], vec_dot, vec3, Wo, r1(bo))
    return (dx, dvecf.reshape(N, 3, HC))


# speedup vs baseline: 1.0125x; 1.0125x over previous
import functools
import math

import jax
import jax.numpy as jnp
import numpy as np
from jax import lax
from jax.experimental import pallas as pl
from jax.experimental.pallas import tpu as pltpu
from jax.experimental.pallas import tpu_sc as plsc

N = 10000
E = 320000
H = 8
D = 16
HC = 128
NRBF = 32
CUTOFF_UPPER = 5.0

BN = 200
BE = 1280

_EPS = 1e-6


def _head_mask():
    r = jax.lax.broadcasted_iota(jnp.int32, (HC, H), 0) // D
    c = jax.lax.broadcasted_iota(jnp.int32, (HC, H), 1)
    return (r == c).astype(jnp.float32)


def _head_mask64():
    r = jax.lax.broadcasted_iota(jnp.int32, (64, H), 0) // (D // 2)
    c = jax.lax.broadcasted_iota(jnp.int32, (64, H), 1)
    return (r == c).astype(jnp.float32)


def _silu(x):
    return x * jax.nn.sigmoid(x)


def _node_body(x_ref, vec_ref, Wq_ref, bq_ref, Wk_ref, bk_ref, Wv_ref, bv_ref,
               Wvec_ref, lns_ref, lnb_ref, lnqs_ref, lnqb_ref, lnks_ref,
               lnkb_ref, q_ref, k_ref, v_ref, vd_ref, v3_ref):
    xb = x_ref[...]
    mean = jnp.mean(xb, axis=1, keepdims=True)
    xc = xb - mean
    var = jnp.mean(xc * xc, axis=1, keepdims=True)
    xn = xc * jax.lax.rsqrt(var + _EPS) * lns_ref[...] + lnb_ref[...]

    MH = _head_mask()

    def headln(z, s, b):
        m = (z @ MH) * (1.0 / D)
        mb = m @ MH.T
        zc = z - mb
        v2 = ((zc * zc) @ MH) * (1.0 / D)
        vb = v2 @ MH.T
        return zc * jax.lax.rsqrt(vb + _EPS) * s + b

    q_ref[...] = headln(xn @ Wq_ref[...] + bq_ref[...], lnqs_ref[...],
                        lnqb_ref[...])
    k_ref[...] = headln(xn @ Wk_ref[...] + bk_ref[...], lnks_ref[...],
                        lnkb_ref[...])
    v_ref[...] = (xn @ Wv_ref[...] + bv_ref[...]).astype(jnp.bfloat16)

    vecb = vec_ref[...]
    Wvec = Wvec_ref[...]
    acc = jnp.zeros((vecb.shape[0], HC), jnp.float32)
    for c in range(3):
        p = vecb[:, c * HC:(c + 1) * HC] @ Wvec
        acc = acc + p[:, :HC] * p[:, HC:2 * HC]
        v3_ref[:, c * HC:(c + 1) * HC] = p[:, 2 * HC:]
    vd_ref[...] = acc


def _node_stage(x, vecf, Wq, bq, Wk, bk, Wv_p, bv_p, Wvec_d, ln_s, ln_b,
                lnq_s, lnq_b, lnk_s, lnk_b):
    nb = N // BN
    row = lambda i: (i, 0)
    rep = lambda i: (0, 0)
    out_shapes = (
        jax.ShapeDtypeStruct((N, HC), jnp.float32),
        jax.ShapeDtypeStruct((N, HC), jnp.float32),
        jax.ShapeDtypeStruct((N, 3 * HC), jnp.bfloat16),
        jax.ShapeDtypeStruct((N, HC), jnp.float32),
        jax.ShapeDtypeStruct((N, 3 * HC), jnp.float32),
    )
    in_specs = [
        pl.BlockSpec((BN, HC), row),
        pl.BlockSpec((BN, 3 * HC), row),
        pl.BlockSpec((HC, HC), rep),
        pl.BlockSpec((1, HC), rep),
        pl.BlockSpec((HC, HC), rep),
        pl.BlockSpec((1, HC), rep),
        pl.BlockSpec((HC, 3 * HC), rep),
        pl.BlockSpec((1, 3 * HC), rep),
        pl.BlockSpec((HC, 3 * HC), rep),
        pl.BlockSpec((1, HC), rep),
        pl.BlockSpec((1, HC), rep),
        pl.BlockSpec((1, HC), rep),
        pl.BlockSpec((1, HC), rep),
        pl.BlockSpec((1, HC), rep),
        pl.BlockSpec((1, HC), rep),
    ]
    out_specs = (
        pl.BlockSpec((BN, HC), row),
        pl.BlockSpec((BN, HC), row),
        pl.BlockSpec((BN, 3 * HC), row),
        pl.BlockSpec((BN, HC), row),
        pl.BlockSpec((BN, 3 * HC), row),
    )
    return pl.pallas_call(
        _node_body, grid=(nb,), in_specs=in_specs, out_specs=out_specs,
        out_shape=out_shapes,
    )(x, vecf, Wq, bq, Wk, bk, Wv_p, bv_p, Wvec_d, ln_s, ln_b, lnq_s, lnq_b,
      lnk_s, lnk_b)


NC = 2
NS = 16
NW = NC * NS
GCH = 80
KVV = 128 + 192 + 192


def _sc_gather(q32, kvv32, senders, receivers, e0, es):
    epw = es // NW
    nch = epw // GCH
    mesh = plsc.VectorSubcoreMesh(core_axis_name="c", subcore_axis_name="s")

    @functools.partial(
        pl.kernel, mesh=mesh,
        out_type=(
            jax.ShapeDtypeStruct((es, HC), jnp.int32),
            jax.ShapeDtypeStruct((es, KVV), jnp.int32),
        ),
        scratch_types=[
            pltpu.VMEM((2, GCH), jnp.int32),
            pltpu.VMEM((2, GCH), jnp.int32),
            pltpu.VMEM((GCH, HC), jnp.int32),
            pltpu.VMEM((GCH, HC), jnp.int32),
            pltpu.VMEM((GCH, KVV), jnp.int32),
            pltpu.VMEM((GCH, KVV), jnp.int32),
            pltpu.SemaphoreType.DMA,
            pltpu.SemaphoreType.DMA,
            pltpu.SemaphoreType.DMA,
            pltpu.SemaphoreType.DMA,
            pltpu.SemaphoreType.DMA,
            pltpu.SemaphoreType.DMA,
        ],
    )
    def gk(q_hbm, kvv_hbm, send_hbm, recv_hbm, qi_hbm, kvvj_hbm,
           idx_r, idx_s, ra0, ra1, rb0, rb1, semi0, semi1, semg0, semg1,
           semw0, semw1):
        wid = lax.axis_index("s") * NC + lax.axis_index("c")
        base0 = wid * epw
        ra = (ra0, ra1)
        rb = (rb0, rb1)
        semi = (semi0, semi1)
        semg = (semg0, semg1)
        semw = (semw0, semw1)

        def body(g, _):
            i0 = 2 * g
            bs = []
            for p in range(2):
                b = pl.multiple_of(base0 + (i0 + p) * GCH, 8)
                src = pl.multiple_of(e0 + b, 8)
                bs.append(b)
                pltpu.async_copy(recv_hbm.at[pl.ds(src, GCH)],
                                 idx_r.at[p], semi[p])
                pltpu.async_copy(send_hbm.at[pl.ds(src, GCH)],
                                 idx_s.at[p], semi[p])
            for p in range(2):
                pltpu.make_async_copy(
                    recv_hbm.at[pl.ds(0, GCH)], idx_r.at[p], semi[p]).wait()
                pltpu.make_async_copy(
                    send_hbm.at[pl.ds(0, GCH)], idx_s.at[p], semi[p]).wait()
                pltpu.async_copy(q_hbm.at[idx_r.at[p]], ra[p], semg[p])
                pltpu.async_copy(kvv_hbm.at[idx_s.at[p]], rb[p], semg[p])
            for p in range(2):
                pltpu.make_async_copy(
                    q_hbm.at[idx_r.at[p]], ra[p], semg[p]).wait()
                pltpu.make_async_copy(
                    kvv_hbm.at[idx_s.at[p]], rb[p], semg[p]).wait()
                pltpu.async_copy(ra[p], qi_hbm.at[pl.ds(bs[p], GCH)],
                                 semw[p])
                pltpu.async_copy(rb[p], kvvj_hbm.at[pl.ds(bs[p], GCH)],
                                 semw[p])
            for p in range(2):
                pltpu.make_async_copy(
                    ra[p], qi_hbm.at[pl.ds(bs[p], GCH)], semw[p]).wait()
                pltpu.make_async_copy(
                    rb[p], kvvj_hbm.at[pl.ds(bs[p], GCH)], semw[p]).wait()
            return ()

        lax.fori_loop(0, nch // 2, body, (), unroll=False)

        if nch % 2:
            b = pl.multiple_of(base0 + (nch - 1) * GCH, 8)
            src = pl.multiple_of(e0 + b, 8)
            pltpu.sync_copy(recv_hbm.at[pl.ds(src, GCH)], idx_r.at[0])
            pltpu.sync_copy(send_hbm.at[pl.ds(src, GCH)], idx_s.at[0])
            pltpu.async_copy(q_hbm.at[idx_r.at[0]], ra0, semg0).wait()
            pltpu.async_copy(kvv_hbm.at[idx_s.at[0]], rb0, semg0).wait()
            pltpu.sync_copy(ra0, qi_hbm.at[pl.ds(b, GCH)])
            pltpu.sync_copy(rb0, kvvj_hbm.at[pl.ds(b, GCH)])

    return gk(q32, kvv32, senders, receivers)


def _unpk(x):
    e = jax.lax.bitcast_convert_type(x << 16, jnp.float32)
    o = jax.lax.bitcast_convert_type(x & jnp.int32(-65536), jnp.float32)
    return e, o


def _msg_body(qi_ref, kvv_ref, ea_ref, evw_ref, Wdk_ref, bdk_ref, Wdv_ref,
              bdv_ref, m_ref):
    ea = ea_ref[...]
    dk = _silu(ea @ Wdk_ref[...] + bdk_ref[...])
    dv = _silu(ea @ Wdv_ref[...] + bdv_ref[...])

    evw = evw_ref[...]
    w = evw[:, 3:4]
    cut = 0.5 * (jnp.cos(w * (math.pi / CUTOFF_UPPER)) + 1.0)
    cut = cut * (w < CUTOFF_UPPER).astype(jnp.float32)

    qi = jax.lax.bitcast_convert_type(qi_ref[...], jnp.float32)
    kvv = kvv_ref[...]
    kj = jax.lax.bitcast_convert_type(kvv[:, :HC], jnp.float32)
    v0, v1 = _unpk(kvv[:, HC:2 * HC])
    v2, c0 = _unpk(kvv[:, 2 * HC:3 * HC])
    c1, c2 = _unpk(kvv[:, 3 * HC:])

    MH = _head_mask()
    t = qi * kj * dk
    attn = _silu(t @ MH) * cut
    ae = attn @ MH.T

    m_ref[0] = v0 * dv[:, :HC] * ae
    A = v1 * dv[:, HC:2 * HC]
    B = v2 * dv[:, 2 * HC:]
    for c, cc in enumerate((c0, c1, c2)):
        m_ref[c + 1] = cc * A + B * evw[:, c:c + 1]


def _msg_stage(qi32, kvvj, edge_attr, evw, Wdk_d, bdk_d, Wdv_d, bdv_d,
               e0, es):
    nb = es // BE
    off = e0 // BE
    row = lambda i: (i, 0)
    erow = lambda i: (i + off, 0)
    rep = lambda i: (0, 0)
    in_specs = [
        pl.BlockSpec((BE, HC), row),
        pl.BlockSpec((BE, KVV), row),
        pl.BlockSpec((BE, NRBF), erow),
        pl.BlockSpec((BE, 4), erow),
        pl.BlockSpec((NRBF, HC), rep),
        pl.BlockSpec((1, HC), rep),
        pl.BlockSpec((NRBF, 3 * HC), rep),
        pl.BlockSpec((1, 3 * HC), rep),
    ]
    out_spec = pl.BlockSpec((4, BE, HC), lambda i: (0, i, 0))
    return pl.pallas_call(
        _msg_body, grid=(nb,), in_specs=in_specs, out_specs=out_spec,
        out_shape=jax.ShapeDtypeStruct((4, es, HC), jnp.float32),
    )(qi32, kvvj, edge_attr, evw, Wdk_d, bdk_d, Wdv_d, bdv_d)


SCH = 80
NPAD = 10240
NPT = NPAD // NS


def _sc_scatter(m_flat, receivers, zeros, e0, es):
    ept = es // NS
    nch = ept // SCH
    mesh = plsc.VectorSubcoreMesh(core_axis_name="c", subcore_axis_name="s")

    assert nch % 2 == 0

    @functools.partial(
        pl.kernel, mesh=mesh,
        out_type=jax.ShapeDtypeStruct((4 * NPAD, HC), jnp.float32),
        scratch_types=[
            pltpu.VMEM_SHARED((NPAD, HC), jnp.float32),
            pltpu.VMEM((SCH,), jnp.int32),
            pltpu.VMEM((SCH,), jnp.int32),
            pltpu.VMEM((SCH, HC), jnp.float32),
            pltpu.VMEM((SCH, HC), jnp.float32),
            pltpu.SemaphoreType.DMA,
            pltpu.SemaphoreType.DMA,
            pltpu.SemaphoreType.DMA,
            pltpu.SemaphoreType.DMA,
        ],
    )
    def sk(m_hbm, recv_hbm, z_hbm, s_hbm, shared, idx0, idx1, rows0, rows1,
           semi0, semi1, semr0, semr1):
        cid = lax.axis_index("c")
        sid = lax.axis_index("s")
        r0 = sid * NPT
        eb0 = sid * ept
        idx = (idx0, idx1)
        rows = (rows0, rows1)
        semi = (semi0, semi1)
        semr = (semr0, semr1)
        for j in range(2):
            ch = cid * 2 + j
            pltpu.sync_copy(z_hbm, shared.at[pl.ds(r0, NPT)])
            plsc.subcore_barrier()

            def start(p, i):
                b = pl.multiple_of(eb0 + i * SCH, 8)
                pltpu.async_copy(
                    recv_hbm.at[pl.ds(pl.multiple_of(e0 + b, 8), SCH)],
                    idx[p], semi[p])
                pltpu.async_copy(
                    m_hbm.at[pl.ds(pl.multiple_of(ch * es + b, 8), SCH)],
                    rows[p], semr[p])

            def fin(p):
                pltpu.make_async_copy(
                    recv_hbm.at[pl.ds(0, SCH)], idx[p], semi[p]).wait()
                pltpu.make_async_copy(
                    m_hbm.at[pl.ds(0, SCH)], rows[p], semr[p]).wait()
                pltpu.sync_copy(rows[p], shared.at[idx[p]], add=True)

            start(0, 0)

            def body(g, _):
                start(1, 2 * g + 1)
                fin(0)
                start(0, 2 * g + 2)
                fin(1)
                return ()

            lax.fori_loop(0, nch // 2 - 1, body, (), unroll=False)
            start(1, nch - 1)
            fin(0)
            fin(1)
            plsc.subcore_barrier()
            pltpu.sync_copy(shared.at[pl.ds(r0, NPT)],
                            s_hbm.at[pl.ds(ch * NPAD + r0, NPT)])
            plsc.subcore_barrier()

    return sk(m_flat, receivers, zeros)


def _out_body(sa_ref, sb_ref, sc_ref, vd_ref, v3_ref, Wo_ref, bo_ref,
              dx_ref, dvec_ref):
    s = sa_ref[...] + sb_ref[...] + sc_ref[...]
    o = s[0] @ Wo_ref[...] + bo_ref[...]
    o1, o2, o3 = o[:, :HC], o[:, HC:2 * HC], o[:, 2 * HC:]
    dx_ref[...] = vd_ref[...] * o2 + o3
    v3 = v3_ref[...]
    for c in range(3):
        dvec_ref[:, c * HC:(c + 1) * HC] = v3[:, c * HC:(c + 1) * HC] * o1 \
            + s[c + 1]


def _out_stage(Sa, Sb, Sc, vec_dot, vec3, Wo_d, bo_d):
    nb = N // BN
    row = lambda i: (i, 0)
    rep = lambda i: (0, 0)
    in_specs = [
        pl.BlockSpec((4, BN, HC), lambda i: (0, i, 0)),
        pl.BlockSpec((4, BN, HC), lambda i: (0, i, 0)),
        pl.BlockSpec((4, BN, HC), lambda i: (0, i, 0)),
        pl.BlockSpec((BN, HC), row),
        pl.BlockSpec((BN, 3 * HC), row),
        pl.BlockSpec((HC, 3 * HC), rep),
        pl.BlockSpec((1, 3 * HC), rep),
    ]
    out_specs = (
        pl.BlockSpec((BN, HC), row),
        pl.BlockSpec((BN, 3 * HC), row),
    )
    return pl.pallas_call(
        _out_body, grid=(nb,), in_specs=in_specs, out_specs=out_specs,
        out_shape=(jax.ShapeDtypeStruct((N, HC), jnp.float32),
                   jax.ShapeDtypeStruct((N, 3 * HC), jnp.float32)),
    )(Sa, Sb, Sc, vec_dot, vec3, Wo_d, bo_d)


def _vperm():
    perm = np.empty((3 * HC,), np.int32)
    for p in range(3):
        for h in range(H):
            for d in range(D):
                perm[p * HC + h * D + d] = h * 3 * D + p * D + d
    return perm


def _pack2(a, b):
    ab = jnp.stack([a, b], axis=-1).reshape(a.shape[0], 2 * a.shape[1])
    return _pack32(ab)


def _pack32(a):
    return jax.lax.bitcast_convert_type(
        a.reshape(a.shape[0], a.shape[1] // 2, 2), jnp.int32)


def kernel(x, vec, edge_weight, edge_attr, edge_vec, ln_s, ln_b, Wq, bq, Wk,
           bk, lnq_s, lnq_b, lnk_s, lnk_b, Wv, bv, Wvec, Wdk, bdk, Wdv, bdv,
           Wo, bo, senders, receivers):
    perm = _vperm()
    Wv_p = Wv[:, perm]
    bv_p = bv[perm].reshape(1, -1)
    Wdk_d = Wdk
    bdk_d = bdk.reshape(1, -1)
    Wdv_d = Wdv[:, perm]
    bdv_d = bdv[perm].reshape(1, -1)

    vecf = vec.reshape(N, 3 * HC)
    evw = jnp.concatenate([edge_vec, edge_weight], axis=1)
    r1 = lambda a: a.reshape(1, -1)
    tile8 = lambda a: jnp.tile(a, (H,)).reshape(1, -1)

    q, k, v, vec_dot, vec3 = _node_stage(
        x, vecf, Wq, r1(bq), Wk, r1(bk), Wv_p, bv_p, Wvec, r1(ln_s),
        r1(ln_b), tile8(lnq_s), tile8(lnq_b), tile8(lnk_s), tile8(lnk_b))

    vb = vecf.astype(jnp.bfloat16)
    q32 = jax.lax.bitcast_convert_type(q, jnp.int32)
    kvv32 = jnp.concatenate(
        [jax.lax.bitcast_convert_type(k, jnp.int32),
         _pack2(v[:, :HC], v[:, HC:2 * HC]),
         _pack2(v[:, 2 * HC:], vb[:, :HC]),
         _pack2(vb[:, HC:2 * HC], vb[:, 2 * HC:])], axis=1)

    zeros = jnp.zeros((NPT, HC), jnp.float32)
    Ss = []
    for e0, es in ((0, 79360), (79360, 120320), (199680, 120320)):
        qi32, kvvj = _sc_gather(q32, kvv32, senders, receivers, e0, es)
        M = _msg_stage(qi32, kvvj, edge_attr, evw, Wdk_d, bdk_d, Wdv_d,
                       bdv_d, e0, es)
        Ss.append(_sc_scatter(M.reshape(4 * es, HC), receivers, zeros,
                              e0, es).reshape(4, NPAD, HC))

    dx, dvecf = _out_stage(Ss[0], Ss[1], Ss[2], vec_dot, vec3, Wo, r1(bo))
    return (dx, dvecf.reshape(N, 3, HC))
